# initial kernel scaffold (unmeasured)
import jax
import jax.numpy as jnp
from jax import lax
from jax.experimental import pallas as pl
from jax.experimental.pallas import tpu as pltpu

N_DEV = 16
SEQ = 256
DM = 1024
HEADS = 8
DH = 128
SCALE = 0.08838834764831843
NPART = 4


def kernel(x, Wq, Wo, Wk, Wv):
    def body(x_ref, wq_ref, wo_ref, wk_ref, wv_ref, out_ref,
             xs_ref, part_ref, pacc_ref,
             x_send_sems, x_recv_sems, p_send_sems, p_recv_sems):
        my = lax.axis_index("i")

        x_sends = []
        for d in range(1, N_DEV):
            tgt = lax.rem(my + d, N_DEV)
            rdma = pltpu.make_async_remote_copy(
                src_ref=x_ref.at[0],
                dst_ref=xs_ref.at[d - 1],
                send_sem=x_send_sems.at[d - 1],
                recv_sem=x_recv_sems.at[d - 1],
                device_id=(tgt,),
                device_id_type=pl.DeviceIdType.MESH,
            )
            rdma.start()
            x_sends.append(rdma)

        def attn_partial(xb):
            q = jnp.dot(xb, wq_ref[...], preferred_element_type=jnp.float32)
            k = jnp.dot(xb, wk_ref[...], preferred_element_type=jnp.float32)
            v = jnp.dot(xb, wv_ref[...], preferred_element_type=jnp.float32)
            heads = []
            for h in range(HEADS):
                sl = slice(h * DH, (h + 1) * DH)
                qh = q[:, sl]
                kh = k[:, sl]
                vh = v[:, sl]
                s = lax.dot_general(
                    qh, kh, (((1,), (1,)), ((), ())),
                    preferred_element_type=jnp.float32,
                ) * SCALE
                m = jnp.max(s, axis=1, keepdims=True)
                p = jnp.exp(s - m)
                l = jnp.sum(p, axis=1, keepdims=True)
                oh = jnp.dot(p, vh, preferred_element_type=jnp.float32) / l
                heads.append(oh)
            ao = jnp.concatenate(heads, axis=1)
            return jnp.dot(ao, wo_ref[...], preferred_element_type=jnp.float32)

        out_ref[0] = attn_partial(x_ref[0])

        p_sends = [None] * N_DEV
        for d in range(1, N_DEV):
            pltpu.make_async_remote_copy(
                src_ref=xs_ref.at[d - 1],
                dst_ref=xs_ref.at[d - 1],
                send_sem=x_recv_sems.at[d - 1],
                recv_sem=x_recv_sems.at[d - 1],
                device_id=(my,),
                device_id_type=pl.DeviceIdType.MESH,
            ).wait_recv()
            slot = (d - 1) % NPART
            if d - NPART >= 1:
                p_sends[d - NPART].wait_send()
            part_ref[slot] = attn_partial(xs_ref[d - 1])
            home = lax.rem(my - d + N_DEV, N_DEV)
            rdma = pltpu.make_async_remote_copy(
                src_ref=part_ref.at[slot],
                dst_ref=pacc_ref.at[d - 1],
                send_sem=p_send_sems.at[d - 1],
                recv_sem=p_recv_sems.at[d - 1],
                device_id=(home,),
                device_id_type=pl.DeviceIdType.MESH,
            )
            rdma.start()
            p_sends[d] = rdma

        for d in range(1, N_DEV):
            pltpu.make_async_remote_copy(
                src_ref=pacc_ref.at[d - 1],
                dst_ref=pacc_ref.at[d - 1],
                send_sem=p_recv_sems.at[d - 1],
                recv_sem=p_recv_sems.at[d - 1],
                device_id=(my,),
                device_id_type=pl.DeviceIdType.MESH,
            ).wait_recv()
            out_ref[0] = out_ref[0] + pacc_ref[d - 1]

        for rdma in x_sends:
            rdma.wait_send()
        for d in range(N_DEV - NPART, N_DEV):
            p_sends[d].wait_send()

    out_shape = jax.ShapeDtypeStruct((1, SEQ, DM), jnp.float32)
    return pl.pallas_call(
        body,
        out_shape=out_shape,
        in_specs=[pl.BlockSpec(memory_space=pltpu.VMEM)] * 5,
        out_specs=pl.BlockSpec(memory_space=pltpu.VMEM),
        scratch_shapes=[
            pltpu.VMEM((N_DEV - 1, SEQ, DM), jnp.float32),
            pltpu.VMEM((NPART, SEQ, DM), jnp.float32),
            pltpu.VMEM((N_DEV - 1, SEQ, DM), jnp.float32),
            pltpu.SemaphoreType.DMA((N_DEV - 1,)),
            pltpu.SemaphoreType.DMA((N_DEV - 1,)),
            pltpu.SemaphoreType.DMA((N_DEV - 1,)),
            pltpu.SemaphoreType.DMA((N_DEV - 1,)),
        ],
    )(x, Wq, Wo, Wk, Wv)


# baseline (device time: 234282 ns/iter reference)
import jax
import jax.numpy as jnp
from jax import lax
from jax.experimental import pallas as pl
from jax.experimental.pallas import tpu as pltpu

N_DEV = 16
SEQ = 256
DM = 1024
HEADS = 8
DH = 128
SCALE = 0.08838834764831843
NPART = 4


def kernel(x, Wq, Wo, Wk, Wv):
    def body(x_ref, wq_ref, wo_ref, wk_ref, wv_ref, out_ref,
             xbf_ref, xs_ref, part_ref, pacc_ref,
             x_send_sems, x_recv_sems, p_send_sems, p_recv_sems):
        my = lax.axis_index("i")

        xbf_ref[...] = x_ref[0].astype(jnp.bfloat16)
        x_sends = []
        for d in range(1, N_DEV):
            tgt = lax.rem(my + d, N_DEV)
            rdma = pltpu.make_async_remote_copy(
                src_ref=xbf_ref,
                dst_ref=xs_ref.at[d - 1],
                send_sem=x_send_sems.at[d - 1],
                recv_sem=x_recv_sems.at[d - 1],
                device_id=(tgt,),
                device_id_type=pl.DeviceIdType.MESH,
            )
            rdma.start()
            x_sends.append(rdma)

        def attn_partial(xb):
            q = jnp.dot(xb, wq_ref[...], preferred_element_type=jnp.float32)
            k = jnp.dot(xb, wk_ref[...], preferred_element_type=jnp.float32)
            v = jnp.dot(xb, wv_ref[...], preferred_element_type=jnp.float32)
            heads = []
            for h in range(HEADS):
                sl = slice(h * DH, (h + 1) * DH)
                qh = q[:, sl]
                kh = k[:, sl]
                vh = v[:, sl]
                s = lax.dot_general(
                    qh, kh, (((1,), (1,)), ((), ())),
                    preferred_element_type=jnp.float32,
                ) * SCALE
                m = jnp.max(s, axis=1, keepdims=True)
                p = jnp.exp(s - m)
                l = jnp.sum(p, axis=1, keepdims=True)
                oh = jnp.dot(p, vh, preferred_element_type=jnp.float32) / l
                heads.append(oh)
            ao = jnp.concatenate(heads, axis=1)
            return jnp.dot(ao, wo_ref[...], preferred_element_type=jnp.float32)

        out_ref[0] = attn_partial(x_ref[0])

        p_sends = [None] * N_DEV
        for d in range(1, N_DEV):
            pltpu.make_async_remote_copy(
                src_ref=xs_ref.at[d - 1],
                dst_ref=xs_ref.at[d - 1],
                send_sem=x_recv_sems.at[d - 1],
                recv_sem=x_recv_sems.at[d - 1],
                device_id=(my,),
                device_id_type=pl.DeviceIdType.MESH,
            ).wait_recv()
            slot = (d - 1) % NPART
            if d - NPART >= 1:
                p_sends[d - NPART].wait_send()
            part_ref[slot] = attn_partial(
                xs_ref[d - 1].astype(jnp.float32)
            ).astype(jnp.bfloat16)
            home = lax.rem(my - d + N_DEV, N_DEV)
            rdma = pltpu.make_async_remote_copy(
                src_ref=part_ref.at[slot],
                dst_ref=pacc_ref.at[d - 1],
                send_sem=p_send_sems.at[d - 1],
                recv_sem=p_recv_sems.at[d - 1],
                device_id=(home,),
                device_id_type=pl.DeviceIdType.MESH,
            )
            rdma.start()
            p_sends[d] = rdma

        for d in range(1, N_DEV):
            pltpu.make_async_remote_copy(
                src_ref=pacc_ref.at[d - 1],
                dst_ref=pacc_ref.at[d - 1],
                send_sem=p_recv_sems.at[d - 1],
                recv_sem=p_recv_sems.at[d - 1],
                device_id=(my,),
                device_id_type=pl.DeviceIdType.MESH,
            ).wait_recv()
            out_ref[0] = out_ref[0] + pacc_ref[d - 1].astype(jnp.float32)

        for rdma in x_sends:
            rdma.wait_send()
        for d in range(N_DEV - NPART, N_DEV):
            p_sends[d].wait_send()

    out_shape = jax.ShapeDtypeStruct((1, SEQ, DM), jnp.float32)
    return pl.pallas_call(
        body,
        out_shape=out_shape,
        in_specs=[pl.BlockSpec(memory_space=pltpu.VMEM)] * 5,
        out_specs=pl.BlockSpec(memory_space=pltpu.VMEM),
        scratch_shapes=[
            pltpu.VMEM((SEQ, DM), jnp.bfloat16),
            pltpu.VMEM((N_DEV - 1, SEQ, DM), jnp.bfloat16),
            pltpu.VMEM((NPART, SEQ, DM), jnp.bfloat16),
            pltpu.VMEM((N_DEV - 1, SEQ, DM), jnp.bfloat16),
            pltpu.SemaphoreType.DMA((N_DEV - 1,)),
            pltpu.SemaphoreType.DMA((N_DEV - 1,)),
            pltpu.SemaphoreType.DMA((N_DEV - 1,)),
            pltpu.SemaphoreType.DMA((N_DEV - 1,)),
        ],
        compiler_params=pltpu.CompilerParams(
            vmem_limit_bytes=44 * 1024 * 1024,
        ),
    )(x, Wq, Wo, Wk, Wv)


# device time: 229265 ns/iter; 1.0219x vs baseline; 1.0219x over previous
import jax
import jax.numpy as jnp
from jax import lax
from jax.experimental import pallas as pl
from jax.experimental.pallas import tpu as pltpu

N_DEV = 16
SEQ = 256
DM = 1024
HEADS = 8
DH = 128
SCALE = 0.08838834764831843
NPART = 4

NEAR_FIRST = sorted(range(1, N_DEV), key=lambda d: (min(d, N_DEV - d), d))
FAR_FIRST = NEAR_FIRST[::-1]


def kernel(x, Wq, Wo, Wk, Wv):
    def body(x_ref, wq_ref, wo_ref, wk_ref, wv_ref, out_ref,
             xbf_ref, wqb_ref, wkb_ref, wvb_ref, wob_ref,
             xs_ref, part_ref, pacc_ref,
             x_send_sems, x_recv_sems, p_send_sems, p_recv_sems):
        my = lax.axis_index("i")

        xbf_ref[...] = x_ref[0].astype(jnp.bfloat16)
        x_sends = []
        for d in FAR_FIRST:
            tgt = lax.rem(my + d, N_DEV)
            rdma = pltpu.make_async_remote_copy(
                src_ref=xbf_ref,
                dst_ref=xs_ref.at[d - 1],
                send_sem=x_send_sems.at[d - 1],
                recv_sem=x_recv_sems.at[d - 1],
                device_id=(tgt,),
                device_id_type=pl.DeviceIdType.MESH,
            )
            rdma.start()
            x_sends.append(rdma)

        wqb_ref[...] = wq_ref[...].astype(jnp.bfloat16)
        wkb_ref[...] = wk_ref[...].astype(jnp.bfloat16)
        wvb_ref[...] = wv_ref[...].astype(jnp.bfloat16)
        wob_ref[...] = wo_ref[...].astype(jnp.bfloat16)

        def attn_partial(xb):
            q = jnp.dot(xb, wqb_ref[...], preferred_element_type=jnp.float32)
            k = jnp.dot(xb, wkb_ref[...], preferred_element_type=jnp.float32)
            v = jnp.dot(xb, wvb_ref[...], preferred_element_type=jnp.float32)
            qb = (q * SCALE).astype(jnp.bfloat16)
            kb = k.astype(jnp.bfloat16)
            vb = v.astype(jnp.bfloat16)
            heads = []
            for h in range(HEADS):
                sl = slice(h * DH, (h + 1) * DH)
                s = lax.dot_general(
                    qb[:, sl], kb[:, sl], (((1,), (1,)), ((), ())),
                    preferred_element_type=jnp.float32,
                )
                m = jnp.max(s, axis=1, keepdims=True)
                p = jnp.exp(s - m)
                l = jnp.sum(p, axis=1, keepdims=True)
                pb = p.astype(jnp.bfloat16)
                oh = jnp.dot(pb, vb[:, sl], preferred_element_type=jnp.float32)
                heads.append((oh / l).astype(jnp.bfloat16))
            ao = jnp.concatenate(heads, axis=1)
            return jnp.dot(ao, wob_ref[...], preferred_element_type=jnp.float32)

        out_ref[0] = attn_partial(xbf_ref[...])

        p_sends = []
        for pos, d in enumerate(NEAR_FIRST):
            pltpu.make_async_remote_copy(
                src_ref=xs_ref.at[d - 1],
                dst_ref=xs_ref.at[d - 1],
                send_sem=x_recv_sems.at[d - 1],
                recv_sem=x_recv_sems.at[d - 1],
                device_id=(my,),
                device_id_type=pl.DeviceIdType.MESH,
            ).wait_recv()
            slot = pos % NPART
            if pos >= NPART:
                p_sends[pos - NPART].wait_send()
            part_ref[slot] = attn_partial(xs_ref[d - 1]).astype(jnp.bfloat16)
            home = lax.rem(my - d + N_DEV, N_DEV)
            rdma = pltpu.make_async_remote_copy(
                src_ref=part_ref.at[slot],
                dst_ref=pacc_ref.at[d - 1],
                send_sem=p_send_sems.at[d - 1],
                recv_sem=p_recv_sems.at[d - 1],
                device_id=(home,),
                device_id_type=pl.DeviceIdType.MESH,
            )
            rdma.start()
            p_sends.append(rdma)

        for d in NEAR_FIRST:
            pltpu.make_async_remote_copy(
                src_ref=pacc_ref.at[d - 1],
                dst_ref=pacc_ref.at[d - 1],
                send_sem=p_recv_sems.at[d - 1],
                recv_sem=p_recv_sems.at[d - 1],
                device_id=(my,),
                device_id_type=pl.DeviceIdType.MESH,
            ).wait_recv()
            out_ref[0] = out_ref[0] + pacc_ref[d - 1].astype(jnp.float32)

        for rdma in x_sends:
            rdma.wait_send()
        for rdma in p_sends[-NPART:]:
            rdma.wait_send()

    out_shape = jax.ShapeDtypeStruct((1, SEQ, DM), jnp.float32)
    return pl.pallas_call(
        body,
        out_shape=out_shape,
        in_specs=[pl.BlockSpec(memory_space=pltpu.VMEM)] * 5,
        out_specs=pl.BlockSpec(memory_space=pltpu.VMEM),
        scratch_shapes=[
            pltpu.VMEM((SEQ, DM), jnp.bfloat16),
            pltpu.VMEM((DM, DM), jnp.bfloat16),
            pltpu.VMEM((DM, DM), jnp.bfloat16),
            pltpu.VMEM((DM, DM), jnp.bfloat16),
            pltpu.VMEM((DM, DM), jnp.bfloat16),
            pltpu.VMEM((N_DEV - 1, SEQ, DM), jnp.bfloat16),
            pltpu.VMEM((NPART, SEQ, DM), jnp.bfloat16),
            pltpu.VMEM((N_DEV - 1, SEQ, DM), jnp.bfloat16),
            pltpu.SemaphoreType.DMA((N_DEV - 1,)),
            pltpu.SemaphoreType.DMA((N_DEV - 1,)),
            pltpu.SemaphoreType.DMA((N_DEV - 1,)),
            pltpu.SemaphoreType.DMA((N_DEV - 1,)),
        ],
        compiler_params=pltpu.CompilerParams(
            vmem_limit_bytes=44 * 1024 * 1024,
        ),
    )(x, Wq, Wo, Wk, Wv)


# device time: 223268 ns/iter; 1.0493x vs baseline; 1.0269x over previous
import jax
import jax.numpy as jnp
from jax import lax
from jax.experimental import pallas as pl
from jax.experimental.pallas import tpu as pltpu

N_DEV = 16
SEQ = 256
DM = 1024
HEADS = 8
DH = 128
SCALE = 0.08838834764831843
NPART = 4

NEAR_FIRST = sorted(range(1, N_DEV), key=lambda d: (min(d, N_DEV - d), d))
FAR_FIRST = NEAR_FIRST[::-1]


def kernel(x, Wq, Wo, Wk, Wv):
    def body(x_ref, wq_ref, wo_ref, wk_ref, wv_ref, out_ref,
             xbf_ref, wqb_ref, wkb_ref, wvb_ref, wob_ref,
             xs_ref, part_ref, pacc_ref,
             x_send_sems, x_recv_sems, p_send_sems, p_recv_sems):
        my = lax.axis_index("i")

        xbf_ref[...] = x_ref[0].astype(jnp.bfloat16)
        x_sends = []
        for d in FAR_FIRST:
            tgt = lax.rem(my + d, N_DEV)
            rdma = pltpu.make_async_remote_copy(
                src_ref=xbf_ref,
                dst_ref=xs_ref.at[d - 1],
                send_sem=x_send_sems.at[d - 1],
                recv_sem=x_recv_sems.at[d - 1],
                device_id=(tgt,),
                device_id_type=pl.DeviceIdType.MESH,
            )
            rdma.start()
            x_sends.append(rdma)

        wqb_ref[...] = wq_ref[...].astype(jnp.bfloat16)
        wkb_ref[...] = wk_ref[...].astype(jnp.bfloat16)
        wvb_ref[...] = wv_ref[...].astype(jnp.bfloat16)
        wob_ref[...] = wo_ref[...].astype(jnp.bfloat16)

        def attn_partial(xb):
            q = jnp.dot(xb, wqb_ref[...], preferred_element_type=jnp.float32)
            k = jnp.dot(xb, wkb_ref[...], preferred_element_type=jnp.float32)
            v = jnp.dot(xb, wvb_ref[...], preferred_element_type=jnp.float32)
            qb = (q * SCALE).astype(jnp.bfloat16)
            kb = k.astype(jnp.bfloat16)
            vb = v.astype(jnp.bfloat16)
            heads = []
            for h in range(HEADS):
                sl = slice(h * DH, (h + 1) * DH)
                s = lax.dot_general(
                    qb[:, sl], kb[:, sl], (((1,), (1,)), ((), ())),
                    preferred_element_type=jnp.float32,
                )
                m = jnp.max(s, axis=1, keepdims=True)
                p = jnp.exp(s - m)
                l = jnp.sum(p, axis=1, keepdims=True)
                pb = p.astype(jnp.bfloat16)
                oh = jnp.dot(pb, vb[:, sl], preferred_element_type=jnp.float32)
                heads.append((oh / l).astype(jnp.bfloat16))
            ao = jnp.concatenate(heads, axis=1)
            return jnp.dot(ao, wob_ref[...], preferred_element_type=jnp.float32)

        out_ref[0] = attn_partial(xbf_ref[...])

        p_sends = []
        for pos, d in enumerate(NEAR_FIRST):
            pltpu.make_async_remote_copy(
                src_ref=xs_ref.at[d - 1],
                dst_ref=xs_ref.at[d - 1],
                send_sem=x_recv_sems.at[d - 1],
                recv_sem=x_recv_sems.at[d - 1],
                device_id=(my,),
                device_id_type=pl.DeviceIdType.MESH,
            ).wait_recv()
            slot = pos
            part_ref[slot] = attn_partial(xs_ref[d - 1]).astype(jnp.bfloat16)
            home = lax.rem(my - d + N_DEV, N_DEV)
            rdma = pltpu.make_async_remote_copy(
                src_ref=part_ref.at[slot],
                dst_ref=pacc_ref.at[d - 1],
                send_sem=p_send_sems.at[d - 1],
                recv_sem=p_recv_sems.at[d - 1],
                device_id=(home,),
                device_id_type=pl.DeviceIdType.MESH,
            )
            rdma.start()
            p_sends.append(rdma)

        for d in NEAR_FIRST:
            pltpu.make_async_remote_copy(
                src_ref=pacc_ref.at[d - 1],
                dst_ref=pacc_ref.at[d - 1],
                send_sem=p_recv_sems.at[d - 1],
                recv_sem=p_recv_sems.at[d - 1],
                device_id=(my,),
                device_id_type=pl.DeviceIdType.MESH,
            ).wait_recv()
            out_ref[0] = out_ref[0] + pacc_ref[d - 1].astype(jnp.float32)

        for rdma in x_sends:
            rdma.wait_send()
        for rdma in p_sends:
            rdma.wait_send()

    out_shape = jax.ShapeDtypeStruct((1, SEQ, DM), jnp.float32)
    return pl.pallas_call(
        body,
        out_shape=out_shape,
        in_specs=[pl.BlockSpec(memory_space=pltpu.VMEM)] * 5,
        out_specs=pl.BlockSpec(memory_space=pltpu.VMEM),
        scratch_shapes=[
            pltpu.VMEM((SEQ, DM), jnp.bfloat16),
            pltpu.VMEM((DM, DM), jnp.bfloat16),
            pltpu.VMEM((DM, DM), jnp.bfloat16),
            pltpu.VMEM((DM, DM), jnp.bfloat16),
            pltpu.VMEM((DM, DM), jnp.bfloat16),
            pltpu.VMEM((N_DEV - 1, SEQ, DM), jnp.bfloat16),
            pltpu.VMEM((N_DEV - 1, SEQ, DM), jnp.bfloat16),
            pltpu.VMEM((N_DEV - 1, SEQ, DM), jnp.bfloat16),
            pltpu.SemaphoreType.DMA((N_DEV - 1,)),
            pltpu.SemaphoreType.DMA((N_DEV - 1,)),
            pltpu.SemaphoreType.DMA((N_DEV - 1,)),
            pltpu.SemaphoreType.DMA((N_DEV - 1,)),
        ],
        compiler_params=pltpu.CompilerParams(
            vmem_limit_bytes=46 * 1024 * 1024,
        ),
    )(x, Wq, Wo, Wk, Wv)


# device time: 163576 ns/iter; 1.4323x vs baseline; 1.3649x over previous
import jax
import jax.numpy as jnp
from jax import lax
from jax.experimental import pallas as pl
from jax.experimental.pallas import tpu as pltpu

N_DEV = 16
HALF = 8
SEQ = 256
DM = 1024
HEADS = 8
DH = 128
SCALE = 0.08838834764831843


def kernel(x, Wq, Wo, Wk, Wv):
    def body(x_ref, wq_ref, wo_ref, wk_ref, wv_ref, out_ref,
             xbf_ref, wqb_ref, wkb_ref, wvb_ref, wob_ref,
             xs_near, xs_far, xs_mir,
             pnear_stage, pred_stage, ownfar_ref, pred_rx, fsum_ref,
             pacc_near, pacc_fs,
             sx_near, rx_near, sx_mir, rx_mir, s_relay, rx_far,
             sp_near, rp_near, sp_red, rp_red, sp_fs, rp_fs):
        my = lax.axis_index("i")
        q = lax.rem(my, HALF)
        base = my - q
        mirror = lax.rem(my + HALF, N_DEV)

        def fwd_mate(k):
            return base + lax.rem(q + k, HALF)

        def back_mate(k):
            return base + lax.rem(q - k + HALF, HALF)

        xbf_ref[...] = x_ref[0].astype(jnp.bfloat16)
        sends = []
        r = pltpu.make_async_remote_copy(
            src_ref=xbf_ref, dst_ref=xs_mir,
            send_sem=sx_mir.at[0], recv_sem=rx_mir.at[0],
            device_id=(mirror,), device_id_type=pl.DeviceIdType.MESH)
        r.start()
        sends.append(r)
        for k in range(1, HALF):
            r = pltpu.make_async_remote_copy(
                src_ref=xbf_ref, dst_ref=xs_near.at[k - 1],
                send_sem=sx_near.at[k - 1], recv_sem=rx_near.at[k - 1],
                device_id=(fwd_mate(k),), device_id_type=pl.DeviceIdType.MESH)
            r.start()
            sends.append(r)

        wqb_ref[...] = wq_ref[...].astype(jnp.bfloat16)
        wkb_ref[...] = wk_ref[...].astype(jnp.bfloat16)
        wvb_ref[...] = wv_ref[...].astype(jnp.bfloat16)
        wob_ref[...] = wo_ref[...].astype(jnp.bfloat16)

        def wait_recv(buf, sem):
            pltpu.make_async_remote_copy(
                src_ref=buf, dst_ref=buf, send_sem=sem, recv_sem=sem,
                device_id=(my,), device_id_type=pl.DeviceIdType.MESH,
            ).wait_recv()

        def attn_partial(xb):
            qm = jnp.dot(xb, wqb_ref[...], preferred_element_type=jnp.float32)
            km = jnp.dot(xb, wkb_ref[...], preferred_element_type=jnp.float32)
            vm = jnp.dot(xb, wvb_ref[...], preferred_element_type=jnp.float32)
            qb = (qm * SCALE).astype(jnp.bfloat16)
            kb = km.astype(jnp.bfloat16)
            vb = vm.astype(jnp.bfloat16)
            heads = []
            for h in range(HEADS):
                sl = slice(h * DH, (h + 1) * DH)
                s = lax.dot_general(qb[:, sl], kb[:, sl],
                                    (((1,), (1,)), ((), ())),
                                    preferred_element_type=jnp.float32)
                m = jnp.max(s, axis=1, keepdims=True)
                p = jnp.exp(s - m)
                l = jnp.sum(p, axis=1, keepdims=True)
                pb = p.astype(jnp.bfloat16)
                oh = jnp.dot(pb, vb[:, sl], preferred_element_type=jnp.float32)
                heads.append((oh / l).astype(jnp.bfloat16))
            ao = jnp.concatenate(heads, axis=1)
            return jnp.dot(ao, wob_ref[...], preferred_element_type=jnp.float32)

        out_ref[0] = attn_partial(xbf_ref[...])

        wait_recv(xs_mir, rx_mir.at[0])
        for k in range(1, HALF):
            r = pltpu.make_async_remote_copy(
                src_ref=xs_mir, dst_ref=xs_far.at[k - 1],
                send_sem=s_relay.at[k - 1], recv_sem=rx_far.at[k - 1],
                device_id=(fwd_mate(k),), device_id_type=pl.DeviceIdType.MESH)
            r.start()
            sends.append(r)
        ownfar_ref[...] = attn_partial(xs_mir[...]).astype(jnp.bfloat16)

        for j in range(1, HALF):
            wait_recv(xs_near.at[j - 1], rx_near.at[j - 1])
            pnear_stage[j - 1] = attn_partial(xs_near[j - 1]).astype(jnp.bfloat16)
            r = pltpu.make_async_remote_copy(
                src_ref=pnear_stage.at[j - 1], dst_ref=pacc_near.at[j - 1],
                send_sem=sp_near.at[j - 1], recv_sem=rp_near.at[j - 1],
                device_id=(back_mate(j),), device_id_type=pl.DeviceIdType.MESH)
            r.start()
            sends.append(r)
            wait_recv(xs_far.at[j - 1], rx_far.at[j - 1])
            pred_stage[j - 1] = attn_partial(xs_far[j - 1]).astype(jnp.bfloat16)
            r = pltpu.make_async_remote_copy(
                src_ref=pred_stage.at[j - 1], dst_ref=pred_rx.at[j - 1],
                send_sem=sp_red.at[j - 1], recv_sem=rp_red.at[j - 1],
                device_id=(back_mate(j),), device_id_type=pl.DeviceIdType.MESH)
            r.start()
            sends.append(r)

        acc = ownfar_ref[...].astype(jnp.float32)
        for j in range(1, HALF):
            wait_recv(pred_rx.at[j - 1], rp_red.at[j - 1])
            acc = acc + pred_rx[j - 1].astype(jnp.float32)
        fsum_ref[...] = acc.astype(jnp.bfloat16)
        r = pltpu.make_async_remote_copy(
            src_ref=fsum_ref, dst_ref=pacc_fs,
            send_sem=sp_fs.at[0], recv_sem=rp_fs.at[0],
            device_id=(mirror,), device_id_type=pl.DeviceIdType.MESH)
        r.start()
        sends.append(r)

        for j in range(1, HALF):
            wait_recv(pacc_near.at[j - 1], rp_near.at[j - 1])
            out_ref[0] = out_ref[0] + pacc_near[j - 1].astype(jnp.float32)
        wait_recv(pacc_fs, rp_fs.at[0])
        out_ref[0] = out_ref[0] + pacc_fs[...].astype(jnp.float32)

        for r in sends:
            r.wait_send()

    out_shape = jax.ShapeDtypeStruct((1, SEQ, DM), jnp.float32)
    n1 = HALF - 1
    return pl.pallas_call(
        body,
        out_shape=out_shape,
        in_specs=[pl.BlockSpec(memory_space=pltpu.VMEM)] * 5,
        out_specs=pl.BlockSpec(memory_space=pltpu.VMEM),
        scratch_shapes=[
            pltpu.VMEM((SEQ, DM), jnp.bfloat16),
            pltpu.VMEM((DM, DM), jnp.bfloat16),
            pltpu.VMEM((DM, DM), jnp.bfloat16),
            pltpu.VMEM((DM, DM), jnp.bfloat16),
            pltpu.VMEM((DM, DM), jnp.bfloat16),
            pltpu.VMEM((n1, SEQ, DM), jnp.bfloat16),
            pltpu.VMEM((n1, SEQ, DM), jnp.bfloat16),
            pltpu.VMEM((SEQ, DM), jnp.bfloat16),
            pltpu.VMEM((n1, SEQ, DM), jnp.bfloat16),
            pltpu.VMEM((n1, SEQ, DM), jnp.bfloat16),
            pltpu.VMEM((SEQ, DM), jnp.bfloat16),
            pltpu.VMEM((n1, SEQ, DM), jnp.bfloat16),
            pltpu.VMEM((SEQ, DM), jnp.bfloat16),
            pltpu.VMEM((n1, SEQ, DM), jnp.bfloat16),
            pltpu.VMEM((SEQ, DM), jnp.bfloat16),
            pltpu.SemaphoreType.DMA((n1,)),
            pltpu.SemaphoreType.DMA((n1,)),
            pltpu.SemaphoreType.DMA((1,)),
            pltpu.SemaphoreType.DMA((1,)),
            pltpu.SemaphoreType.DMA((n1,)),
            pltpu.SemaphoreType.DMA((n1,)),
            pltpu.SemaphoreType.DMA((n1,)),
            pltpu.SemaphoreType.DMA((n1,)),
            pltpu.SemaphoreType.DMA((n1,)),
            pltpu.SemaphoreType.DMA((n1,)),
            pltpu.SemaphoreType.DMA((1,)),
            pltpu.SemaphoreType.DMA((1,)),
        ],
        compiler_params=pltpu.CompilerParams(
            vmem_limit_bytes=46 * 1024 * 1024,
        ),
    )(x, Wq, Wo, Wk, Wv)


# device time: 158738 ns/iter; 1.4759x vs baseline; 1.0305x over previous
import jax
import jax.numpy as jnp
from jax import lax
from jax.experimental import pallas as pl
from jax.experimental.pallas import tpu as pltpu

N_DEV = 16
HALF = 8
SEQ = 256
DM = 1024
HEADS = 8
DH = 128
SCALE = 0.08838834764831843


def kernel(x, Wq, Wo, Wk, Wv):
    def body(x_ref, wq_ref, wo_ref, wk_ref, wv_ref, out_ref,
             xbf_ref, wqb_ref, wkb_ref, wvb_ref, wob_ref,
             xs_near, xs_far, xs_mir,
             combo_stage, ownfar_ref, combo_rx, fsum_ref, pacc_fs,
             sx_near, rx_near, sx_mir, rx_mir, s_relay, rx_far,
             sp_comb, rp_comb, sp_fs, rp_fs):
        my = lax.axis_index("i")
        q = lax.rem(my, HALF)
        base = my - q
        mirror = lax.rem(my + HALF, N_DEV)

        def fwd_mate(k):
            return base + lax.rem(q + k, HALF)

        def back_mate(k):
            return base + lax.rem(q - k + HALF, HALF)

        barrier_sem = pltpu.get_barrier_semaphore()
        for k in range(1, HALF):
            pl.semaphore_signal(barrier_sem, inc=1, device_id=(fwd_mate(k),),
                                device_id_type=pl.DeviceIdType.MESH)
        pl.semaphore_signal(barrier_sem, inc=1, device_id=(mirror,),
                            device_id_type=pl.DeviceIdType.MESH)
        pl.semaphore_wait(barrier_sem, HALF)

        xbf_ref[...] = x_ref[0].astype(jnp.bfloat16)
        sends = []
        r = pltpu.make_async_remote_copy(
            src_ref=xbf_ref, dst_ref=xs_mir,
            send_sem=sx_mir.at[0], recv_sem=rx_mir.at[0],
            device_id=(mirror,), device_id_type=pl.DeviceIdType.MESH)
        r.start()
        sends.append(r)
        for k in range(1, HALF):
            r = pltpu.make_async_remote_copy(
                src_ref=xbf_ref, dst_ref=xs_near.at[k - 1],
                send_sem=sx_near.at[k - 1], recv_sem=rx_near.at[k - 1],
                device_id=(fwd_mate(k),), device_id_type=pl.DeviceIdType.MESH)
            r.start()
            sends.append(r)

        wqb_ref[...] = wq_ref[...].astype(jnp.bfloat16)
        wkb_ref[...] = wk_ref[...].astype(jnp.bfloat16)
        wvb_ref[...] = wv_ref[...].astype(jnp.bfloat16)
        wob_ref[...] = wo_ref[...].astype(jnp.bfloat16)

        def wait_recv(buf, sem):
            pltpu.make_async_remote_copy(
                src_ref=buf, dst_ref=buf, send_sem=sem, recv_sem=sem,
                device_id=(my,), device_id_type=pl.DeviceIdType.MESH,
            ).wait_recv()

        def attn_partial(xb):
            qm = jnp.dot(xb, wqb_ref[...], preferred_element_type=jnp.float32)
            km = jnp.dot(xb, wkb_ref[...], preferred_element_type=jnp.float32)
            vm = jnp.dot(xb, wvb_ref[...], preferred_element_type=jnp.float32)
            qb = (qm * SCALE).astype(jnp.bfloat16)
            kb = km.astype(jnp.bfloat16)
            vb = vm.astype(jnp.bfloat16)
            heads = []
            for h in range(HEADS):
                sl = slice(h * DH, (h + 1) * DH)
                s = lax.dot_general(qb[:, sl], kb[:, sl],
                                    (((1,), (1,)), ((), ())),
                                    preferred_element_type=jnp.float32)
                m = jnp.max(s, axis=1, keepdims=True)
                p = jnp.exp(s - m)
                l = jnp.sum(p, axis=1, keepdims=True)
                pb = p.astype(jnp.bfloat16)
                oh = jnp.dot(pb, vb[:, sl], preferred_element_type=jnp.float32)
                heads.append((oh / l).astype(jnp.bfloat16))
            ao = jnp.concatenate(heads, axis=1)
            return jnp.dot(ao, wob_ref[...], preferred_element_type=jnp.float32)

        out_ref[0] = attn_partial(xbf_ref[...])

        wait_recv(xs_mir, rx_mir.at[0])
        for k in range(1, HALF):
            r = pltpu.make_async_remote_copy(
                src_ref=xs_mir, dst_ref=xs_far.at[k - 1],
                send_sem=s_relay.at[k - 1], recv_sem=rx_far.at[k - 1],
                device_id=(fwd_mate(k),), device_id_type=pl.DeviceIdType.MESH)
            r.start()
            sends.append(r)
        ownfar_ref[...] = attn_partial(xs_mir[...]).astype(jnp.bfloat16)

        for j in range(1, HALF):
            wait_recv(xs_near.at[j - 1], rx_near.at[j - 1])
            combo_stage[j - 1, 0] = attn_partial(xs_near[j - 1]).astype(jnp.bfloat16)
            wait_recv(xs_far.at[j - 1], rx_far.at[j - 1])
            combo_stage[j - 1, 1] = attn_partial(xs_far[j - 1]).astype(jnp.bfloat16)
            r = pltpu.make_async_remote_copy(
                src_ref=combo_stage.at[j - 1], dst_ref=combo_rx.at[j - 1],
                send_sem=sp_comb.at[j - 1], recv_sem=rp_comb.at[j - 1],
                device_id=(back_mate(j),), device_id_type=pl.DeviceIdType.MESH)
            r.start()
            sends.append(r)

        acc = ownfar_ref[...].astype(jnp.float32)
        for j in range(1, HALF):
            wait_recv(combo_rx.at[j - 1], rp_comb.at[j - 1])
            acc = acc + combo_rx[j - 1, 1].astype(jnp.float32)
        fsum_ref[...] = acc.astype(jnp.bfloat16)
        r = pltpu.make_async_remote_copy(
            src_ref=fsum_ref, dst_ref=pacc_fs,
            send_sem=sp_fs.at[0], recv_sem=rp_fs.at[0],
            device_id=(mirror,), device_id_type=pl.DeviceIdType.MESH)
        r.start()
        sends.append(r)
        for j in range(1, HALF):
            out_ref[0] = out_ref[0] + combo_rx[j - 1, 0].astype(jnp.float32)

        wait_recv(pacc_fs, rp_fs.at[0])
        out_ref[0] = out_ref[0] + pacc_fs[...].astype(jnp.float32)

        for r in sends:
            r.wait_send()

    out_shape = jax.ShapeDtypeStruct((1, SEQ, DM), jnp.float32)
    n1 = HALF - 1
    return pl.pallas_call(
        body,
        out_shape=out_shape,
        in_specs=[pl.BlockSpec(memory_space=pltpu.VMEM)] * 5,
        out_specs=pl.BlockSpec(memory_space=pltpu.VMEM),
        scratch_shapes=[
            pltpu.VMEM((SEQ, DM), jnp.bfloat16),
            pltpu.VMEM((DM, DM), jnp.bfloat16),
            pltpu.VMEM((DM, DM), jnp.bfloat16),
            pltpu.VMEM((DM, DM), jnp.bfloat16),
            pltpu.VMEM((DM, DM), jnp.bfloat16),
            pltpu.VMEM((n1, SEQ, DM), jnp.bfloat16),
            pltpu.VMEM((n1, SEQ, DM), jnp.bfloat16),
            pltpu.VMEM((SEQ, DM), jnp.bfloat16),
            pltpu.VMEM((n1, 2, SEQ, DM), jnp.bfloat16),
            pltpu.VMEM((SEQ, DM), jnp.bfloat16),
            pltpu.VMEM((n1, 2, SEQ, DM), jnp.bfloat16),
            pltpu.VMEM((SEQ, DM), jnp.bfloat16),
            pltpu.VMEM((SEQ, DM), jnp.bfloat16),
            pltpu.SemaphoreType.DMA((n1,)),
            pltpu.SemaphoreType.DMA((n1,)),
            pltpu.SemaphoreType.DMA((1,)),
            pltpu.SemaphoreType.DMA((1,)),
            pltpu.SemaphoreType.DMA((n1,)),
            pltpu.SemaphoreType.DMA((n1,)),
            pltpu.SemaphoreType.DMA((n1,)),
            pltpu.SemaphoreType.DMA((n1,)),
            pltpu.SemaphoreType.DMA((1,)),
            pltpu.SemaphoreType.DMA((1,)),
        ],
        compiler_params=pltpu.CompilerParams(
            vmem_limit_bytes=46 * 1024 * 1024,
            collective_id=0,
        ),
    )(x, Wq, Wo, Wk, Wv)
